# Initial kernel scaffold; baseline (speedup 1.0000x reference)
import jax
import jax.numpy as jnp
from jax.experimental import pallas as pl


def kernel(x, edge_index, W1, b1, W2, b2, Wz, bz, Wr, br, Wc, bc, Wo, bo):
    raise NotImplementedError("write your pallas kernel here")


# speedup vs baseline: 5.6405x; 5.6405x over previous
import functools

import jax
import jax.numpy as jnp
from jax import lax
from jax.experimental import pallas as pl
from jax.experimental.pallas import tpu as pltpu
from jax.experimental.pallas import tpu_sc as plsc

N_NODES = 50000
N_EDGES = 800000
HID = 64
SEQ = 12
HORIZON = 12

RB = 256
NP = 50176
NBR = NP // RB
BLK = 128
EP = 802816
NBLK = EP // BLK
N_TILES = 16
ACC_R = 51200
ROWS_PT = ACC_R // N_TILES
GARB = NP

_MESH = functools.partial(
    plsc.VectorSubcoreMesh, core_axis_name="c", subcore_axis_name="s")
_SC_PARAMS = pltpu.CompilerParams(use_tc_tiling_on_sc=False)


def _fill_rows(ref, rows, wh, value):
    vec = jnp.full((16,), value, jnp.float32)

    def body(i, _):
        for j in range(wh // 16):
            ref[i, pl.ds(16 * j, 16)] = vec
        return 0

    lax.fori_loop(0, rows, body, 0)


def _zero_acc(zrow, acc, tile):
    base = tile * ROWS_PT

    def body(i, _):
        pltpu.sync_copy(zrow, acc.at[pl.ds(base + i * BLK, BLK), :])
        return 0

    lax.fori_loop(0, ROWS_PT // BLK, body, 0)


def _copy_out(acc, out, core, tile, wh):
    base = tile * ROWS_PT
    nch = jnp.minimum(ROWS_PT // BLK, (NP - base) // BLK)

    def body(i, _):
        r0 = base + i * BLK
        pltpu.sync_copy(acc.at[pl.ds(r0, BLK), :], out.at[core, pl.ds(r0, BLK), :])
        return 0

    lax.fori_loop(0, nch, body, 0)


def _make_spmm(wh):
    nbt = NBLK // N_TILES

    @functools.partial(
        pl.kernel,
        out_type=jax.ShapeDtypeStruct((2, NP, wh), jnp.float32),
        mesh=_MESH(),
        scratch_types=[
            pltpu.VMEM((BLK,), jnp.int32),
            pltpu.VMEM((BLK,), jnp.int32),
            pltpu.VMEM((BLK, wh), jnp.float32),
            pltpu.VMEM((BLK, wh), jnp.float32),
            pltpu.SemaphoreType.DMA,
            pltpu.VMEM_SHARED((ACC_R, wh), jnp.float32),
        ],
        compiler_params=_SC_PARAMS,
    )
    def spmm(table, srcb, dstb, out, src_v, dst_v, gbuf, zrow, sem, acc):
        c = lax.axis_index("c")
        s = lax.axis_index("s")
        _fill_rows(zrow, BLK, wh, 0.0)
        _zero_acc(zrow, acc, s)
        plsc.subcore_barrier()

        blk0 = s * nbt
        off = c * NP

        def body(b, _):
            blk = blk0 + b
            pltpu.sync_copy(srcb.at[blk], src_v)
            pltpu.sync_copy(dstb.at[blk], dst_v)
            for j in range(BLK // 16):
                sl = pl.ds(16 * j, 16)
                src_v[sl] = src_v[sl] + off
            pltpu.async_copy(table.at[src_v], gbuf, sem).wait()
            pltpu.sync_copy(gbuf, acc.at[dst_v], add=True)
            return 0

        lax.fori_loop(0, nbt, body, 0)
        plsc.subcore_barrier()
        _copy_out(acc, out, c, s, wh)

    return spmm


_spmm32 = _make_spmm(32)
_spmm16 = _make_spmm(16)


@functools.partial(
    pl.kernel,
    out_type=jax.ShapeDtypeStruct((2, NP, 16), jnp.float32),
    mesh=_MESH(),
    scratch_types=[
        pltpu.VMEM((BLK,), jnp.int32),
        pltpu.VMEM((BLK, 16), jnp.float32),
        pltpu.VMEM((BLK, 16), jnp.float32),
        pltpu.VMEM_SHARED((ACC_R, 16), jnp.float32),
    ],
    compiler_params=_SC_PARAMS,
)
def _deg_kernel(srcsb, out, idx_v, ones, zrow, acc):
    c = lax.axis_index("c")
    s = lax.axis_index("s")
    _fill_rows(ones, BLK, 16, 1.0)
    _fill_rows(zrow, BLK, 16, 0.0)
    _zero_acc(zrow, acc, s)
    plsc.subcore_barrier()

    nbt = NBLK // (2 * N_TILES)
    blk0 = (c * N_TILES + s) * nbt

    def body(b, _):
        pltpu.sync_copy(srcsb.at[blk0 + b], idx_v)
        pltpu.sync_copy(ones, acc.at[idx_v], add=True)
        return 0

    lax.fori_loop(0, nbt, body, 0)
    plsc.subcore_barrier()
    _copy_out(acc, out, c, s, 16)


def _prep_body(degs, xp, dinv_o, ux_o):
    deg = degs[0, :, 0:1] + degs[1, :, 0:1]
    di = jnp.where(deg > 0, lax.rsqrt(deg), 0.0)
    dinv_o[...] = di
    xb = xp[...]
    ux_o[...] = jnp.stack([di * xb[:, :16], di * xb[:, 16:]], axis=0)


def _tc_prep(degs, xp):
    return pl.pallas_call(
        _prep_body,
        grid=(NBR,),
        in_specs=[
            pl.BlockSpec((2, RB, 16), lambda i: (0, i, 0)),
            pl.BlockSpec((RB, 32), lambda i: (i, 0)),
        ],
        out_specs=[
            pl.BlockSpec((RB, 1), lambda i: (i, 0)),
            pl.BlockSpec((2, RB, 16), lambda i: (0, i, 0)),
        ],
        out_shape=[
            jax.ShapeDtypeStruct((NP, 1), jnp.float32),
            jax.ShapeDtypeStruct((2, NP, 16), jnp.float32),
        ],
    )(degs, xp)


def _mid_body(s1, dinv, out):
    di = dinv[...]
    out[...] = (di * di) * s1[...]


def _tc_mid(s1, dinv, wh):
    return pl.pallas_call(
        _mid_body,
        grid=(NBR,),
        in_specs=[
            pl.BlockSpec((2, RB, wh), lambda i: (0, i, 0)),
            pl.BlockSpec((RB, 1), lambda i: (i, 0)),
        ],
        out_specs=pl.BlockSpec((2, RB, wh), lambda i: (0, i, 0)),
        out_shape=jax.ShapeDtypeStruct((2, NP, wh), jnp.float32),
    )(s1, dinv)


def _xcat_body(xp, s1x, s2x, dinv, out):
    di = dinv[...]
    xb = xp[...]
    s1 = s1x[...]
    s2 = s2x[...]
    px = -di * jnp.concatenate([s1[0], s1[1]], axis=1)
    p2x = di * jnp.concatenate([s2[0], s2[1]], axis=1)
    zeros = jnp.zeros((RB, 2), jnp.float32)
    rows = [
        jnp.concatenate(
            [xb[:, 2 * t:2 * t + 2], px[:, 2 * t:2 * t + 2],
             p2x[:, 2 * t:2 * t + 2], zeros], axis=1)
        for t in range(SEQ)
    ]
    out[...] = jnp.stack(rows, axis=0)


def _tc_xcat(xp, s1x, s2x, dinv):
    return pl.pallas_call(
        _xcat_body,
        grid=(NBR,),
        in_specs=[
            pl.BlockSpec((RB, 32), lambda i: (i, 0)),
            pl.BlockSpec((2, RB, 16), lambda i: (0, i, 0)),
            pl.BlockSpec((2, RB, 16), lambda i: (0, i, 0)),
            pl.BlockSpec((RB, 1), lambda i: (i, 0)),
        ],
        out_specs=pl.BlockSpec((SEQ, RB, 8), lambda i: (0, i, 0)),
        out_shape=jax.ShapeDtypeStruct((SEQ, NP, 8), jnp.float32),
    )(xp, s1x, s2x, dinv)


def _step_body(h_r, s1_r, s2_r, xc_r, dinv_r, wcat_r, b1_r, h3_r, b2_r,
               wz_r, bz_r, wr_r, br_r, wc_r, bc_r, hn_o, u_o):
    di = dinv_r[...]
    h = h_r[...]
    s1 = s1_r[...]
    s2 = s2_r[...]
    s1f = jnp.concatenate([s1[0], s1[1]], axis=1)
    s2f = jnp.concatenate([s2[0], s2[1]], axis=1)

    ic = jnp.dot(xc_r[...], wcat_r[...],
                 preferred_element_type=jnp.float32) + b1_r[...]
    hcat = jnp.concatenate([h, -di * s1f, di * s2f], axis=1)
    hc = jnp.dot(hcat, h3_r[...], preferred_element_type=jnp.float32) + b2_r[...]

    g = jnp.concatenate([ic, hc], axis=1)
    z = jax.nn.sigmoid(jnp.dot(g, wz_r[...],
                               preferred_element_type=jnp.float32) + bz_r[...])
    r = jax.nn.sigmoid(jnp.dot(g, wr_r[...],
                               preferred_element_type=jnp.float32) + br_r[...])
    cand = jnp.concatenate([ic, r * hc], axis=1)
    ht = jnp.tanh(jnp.dot(cand, wc_r[...],
                          preferred_element_type=jnp.float32) + bc_r[...])
    hn = z * h + (1.0 - z) * ht
    hn_o[...] = hn
    u_o[...] = jnp.stack([di * hn[:, :32], di * hn[:, 32:]], axis=0)


def _full(shape):
    return pl.BlockSpec(shape, lambda i: tuple(0 for _ in shape))


def _tc_step(h, s1h, s2h, xct, dinv, wcat, b1, h3, b2, wz, bz, wr, br, wc, bc):
    return pl.pallas_call(
        _step_body,
        grid=(NBR,),
        in_specs=[
            pl.BlockSpec((RB, HID), lambda i: (i, 0)),
            pl.BlockSpec((2, RB, 32), lambda i: (0, i, 0)),
            pl.BlockSpec((2, RB, 32), lambda i: (0, i, 0)),
            pl.BlockSpec((RB, 8), lambda i: (i, 0)),
            pl.BlockSpec((RB, 1), lambda i: (i, 0)),
            _full((8, HID)), _full((1, HID)),
            _full((3 * HID, HID)), _full((1, HID)),
            _full((2 * HID, HID)), _full((1, HID)),
            _full((2 * HID, HID)), _full((1, HID)),
            _full((2 * HID, HID)), _full((1, HID)),
        ],
        out_specs=[
            pl.BlockSpec((RB, HID), lambda i: (i, 0)),
            pl.BlockSpec((2, RB, 32), lambda i: (0, i, 0)),
        ],
        out_shape=[
            jax.ShapeDtypeStruct((NP, HID), jnp.float32),
            jax.ShapeDtypeStruct((2, NP, 32), jnp.float32),
        ],
    )(h, s1h, s2h, xct, dinv, wcat, b1, h3, b2, wz, bz, wr, br, wc, bc)


def _epi_body(h_r, wo_r, bo_r, out_o):
    out_o[...] = jnp.dot(h_r[...], wo_r[...],
                         preferred_element_type=jnp.float32) + bo_r[...]


def _tc_epi(h, wo, bo):
    return pl.pallas_call(
        _epi_body,
        grid=(NBR,),
        in_specs=[
            pl.BlockSpec((RB, HID), lambda i: (i, 0)),
            _full((HID, HORIZON)), _full((1, HORIZON)),
        ],
        out_specs=pl.BlockSpec((RB, HORIZON), lambda i: (i, 0)),
        out_shape=jax.ShapeDtypeStruct((NP, HORIZON), jnp.float32),
    )(h, wo, bo)


def kernel(x, edge_index, W1, b1, W2, b2, Wz, bz, Wr, br, Wc, bc, Wo, bo):
    f32 = jnp.float32
    src = edge_index[0].astype(jnp.int32)
    dst = edge_index[1].astype(jnp.int32)
    pad = EP - N_EDGES
    srcb = jnp.pad(src, (0, pad)).reshape(NBLK, BLK)
    dstb = jnp.pad(dst, (0, pad), constant_values=GARB).reshape(NBLK, BLK)
    srcsb = jnp.pad(src, (0, pad), constant_values=GARB).reshape(NBLK, BLK)

    xp = jnp.transpose(x[0], (1, 0, 2)).reshape(N_NODES, SEQ * 2)
    xp = jnp.pad(xp, ((0, NP - N_NODES), (0, 32 - SEQ * 2)))

    wcat = jnp.concatenate(
        [W1[0] - W1[2], W1[1], 2.0 * W1[2], jnp.zeros((2, HID), f32)], axis=0)
    h3 = jnp.concatenate([W2[0] - W2[2], W2[1], 2.0 * W2[2]], axis=0)
    b1r = b1.reshape(1, HID)
    b2r = b2.reshape(1, HID)
    bzr = bz.reshape(1, HID)
    brr = br.reshape(1, HID)
    bcr = bc.reshape(1, HID)
    bor = bo.reshape(1, HORIZON)

    degs = _deg_kernel(srcsb)
    dinv, ux = _tc_prep(degs, xp)

    s1x = _spmm16(ux.reshape(2 * NP, 16), srcb, dstb)
    u2x = _tc_mid(s1x, dinv, 16)
    s2x = _spmm16(u2x.reshape(2 * NP, 16), srcb, dstb)

    xcat = _tc_xcat(xp, s1x, s2x, dinv)

    h = jnp.zeros((NP, HID), f32)
    zs = jnp.zeros((2, NP, 32), f32)
    u = None
    for t in range(SEQ):
        if t == 0:
            s1h, s2h = zs, zs
        else:
            s1h = _spmm32(u.reshape(2 * NP, 32), srcb, dstb)
            u2h = _tc_mid(s1h, dinv, 32)
            s2h = _spmm32(u2h.reshape(2 * NP, 32), srcb, dstb)
        h, u = _tc_step(h, s1h, s2h, xcat[t], dinv, wcat, b1r, h3, b2r,
                        Wz, bzr, Wr, brr, Wc, bcr)

    out = _tc_epi(h, Wo, bor)
    return out[:N_NODES].T.reshape(1, HORIZON, N_NODES)


# speedup vs baseline: 11.2924x; 2.0020x over previous
import functools

import jax
import jax.numpy as jnp
from jax import lax
from jax.experimental import pallas as pl
from jax.experimental.pallas import tpu as pltpu
from jax.experimental.pallas import tpu_sc as plsc

N_NODES = 50000
N_EDGES = 800000
HID = 64
SEQ = 12
HORIZON = 12

RB = 256
NP = 50176
NBR = NP // RB
BLK = 128
EP = 802816
NBLK = EP // BLK
N_TILES = 16
ACC_R = 51200
ROWS_PT = ACC_R // N_TILES
GARB = NP

_MESH = functools.partial(
    plsc.VectorSubcoreMesh, core_axis_name="c", subcore_axis_name="s")
_SC_PARAMS = pltpu.CompilerParams(use_tc_tiling_on_sc=False)


def _fill_rows(ref, rows, wh, value):
    vec = jnp.full((16,), value, jnp.float32)

    def body(i, _):
        for j in range(wh // 16):
            ref[i, pl.ds(16 * j, 16)] = vec
        return 0

    lax.fori_loop(0, rows, body, 0)


def _zero_acc(zrow, acc, tile):
    base = tile * ROWS_PT

    def body(i, _):
        pltpu.sync_copy(zrow, acc.at[pl.ds(base + i * BLK, BLK), :])
        return 0

    lax.fori_loop(0, ROWS_PT // BLK, body, 0)


def _copy_out(acc, out, core, tile, wh):
    base = tile * ROWS_PT
    nch = jnp.minimum(ROWS_PT // BLK, (NP - base) // BLK)

    def body(i, _):
        r0 = base + i * BLK
        pltpu.sync_copy(acc.at[pl.ds(r0, BLK), :], out.at[core, pl.ds(r0, BLK), :])
        return 0

    lax.fori_loop(0, nch, body, 0)


GRP = 14


def _make_spmm(wh, grp):
    nbt = NBLK // N_TILES
    nfull, tail = divmod(nbt, grp)

    @functools.partial(
        pl.kernel,
        out_type=jax.ShapeDtypeStruct((2, NP, wh), jnp.float32),
        mesh=_MESH(),
        scratch_types=[
            pltpu.VMEM((grp, BLK), jnp.int32),
            pltpu.VMEM((grp, BLK), jnp.int32),
            pltpu.VMEM((grp * BLK, wh), jnp.float32),
            pltpu.SemaphoreType.DMA,
            pltpu.SemaphoreType.DMA,
            pltpu.VMEM_SHARED((ACC_R, wh), jnp.float32),
        ],
        compiler_params=_SC_PARAMS,
    )
    def spmm(table, src3, dstb, out, sidx, didx, gbuf, gsem, ssem, acc):
        c = lax.axis_index("c")
        s = lax.axis_index("s")
        _fill_rows(gbuf, BLK, wh, 0.0)
        _zero_acc(gbuf.at[pl.ds(0, BLK)], acc, s)
        plsc.subcore_barrier()

        blk0 = s * nbt

        def run_group(row0, n):
            pltpu.sync_copy(src3.at[c, pl.ds(row0, n)], sidx.at[pl.ds(0, n)])
            pltpu.sync_copy(dstb.at[pl.ds(row0, n)], didx.at[pl.ds(0, n)])
            gds = [
                pltpu.async_copy(
                    table.at[sidx.at[g]], gbuf.at[pl.ds(g * BLK, BLK)], gsem)
                for g in range(n)
            ]
            sds = []
            for g in range(n):
                gds[g].wait()
                sds.append(pltpu.async_copy(
                    gbuf.at[pl.ds(g * BLK, BLK)], acc.at[didx.at[g]], ssem,
                    add=True))
            for d in sds:
                d.wait()

        def grp_body(gi, _):
            run_group(blk0 + gi * grp, grp)
            return 0

        lax.fori_loop(0, nfull, grp_body, 0)
        if tail:
            run_group(blk0 + nfull * grp, tail)
        plsc.subcore_barrier()
        _copy_out(acc, out, c, s, wh)

    return spmm


_spmm32 = _make_spmm(32, 6)
_spmm16 = _make_spmm(16, GRP)


@functools.partial(
    pl.kernel,
    out_type=jax.ShapeDtypeStruct((2, NP, 16), jnp.float32),
    mesh=_MESH(),
    scratch_types=[
        pltpu.VMEM((GRP, BLK), jnp.int32),
        pltpu.VMEM((BLK, 16), jnp.float32),
        pltpu.VMEM((BLK, 16), jnp.float32),
        pltpu.SemaphoreType.DMA,
        pltpu.VMEM_SHARED((ACC_R, 16), jnp.float32),
    ],
    compiler_params=_SC_PARAMS,
)
def _deg_kernel(srcsb, out, didx, ones, zrow, sem, acc):
    c = lax.axis_index("c")
    s = lax.axis_index("s")
    _fill_rows(ones, BLK, 16, 1.0)
    _fill_rows(zrow, BLK, 16, 0.0)
    _zero_acc(zrow, acc, s)
    plsc.subcore_barrier()

    nbt = NBLK // (2 * N_TILES)
    blk0 = (c * N_TILES + s) * nbt

    def grp_body(gi, _):
        row0 = blk0 + gi * GRP
        pltpu.sync_copy(srcsb.at[pl.ds(row0, GRP)], didx)
        sds = [
            pltpu.async_copy(ones, acc.at[didx.at[g]], sem, add=True)
            for g in range(GRP)
        ]
        for d in sds:
            d.wait()
        return 0

    lax.fori_loop(0, nbt // GRP, grp_body, 0)
    plsc.subcore_barrier()
    _copy_out(acc, out, c, s, 16)


def _prep_body(degs, xp, dinv_o, ux_o):
    deg = degs[0, :, 0:1] + degs[1, :, 0:1]
    di = jnp.where(deg > 0, lax.rsqrt(deg), 0.0)
    dinv_o[...] = di
    xb = xp[...]
    ux_o[...] = jnp.stack([di * xb[:, :16], di * xb[:, 16:]], axis=0)


def _tc_prep(degs, xp):
    return pl.pallas_call(
        _prep_body,
        grid=(NBR,),
        in_specs=[
            pl.BlockSpec((2, RB, 16), lambda i: (0, i, 0)),
            pl.BlockSpec((RB, 32), lambda i: (i, 0)),
        ],
        out_specs=[
            pl.BlockSpec((RB, 1), lambda i: (i, 0)),
            pl.BlockSpec((2, RB, 16), lambda i: (0, i, 0)),
        ],
        out_shape=[
            jax.ShapeDtypeStruct((NP, 1), jnp.float32),
            jax.ShapeDtypeStruct((2, NP, 16), jnp.float32),
        ],
    )(degs, xp)


def _mid_body(s1, dinv, out):
    di = dinv[...]
    out[...] = (di * di) * s1[...]


def _tc_mid(s1, dinv, wh):
    return pl.pallas_call(
        _mid_body,
        grid=(NBR,),
        in_specs=[
            pl.BlockSpec((2, RB, wh), lambda i: (0, i, 0)),
            pl.BlockSpec((RB, 1), lambda i: (i, 0)),
        ],
        out_specs=pl.BlockSpec((2, RB, wh), lambda i: (0, i, 0)),
        out_shape=jax.ShapeDtypeStruct((2, NP, wh), jnp.float32),
    )(s1, dinv)


def _xcat_body(xp, s1x, s2x, dinv, out):
    di = dinv[...]
    xb = xp[...]
    s1 = s1x[...]
    s2 = s2x[...]
    px = -di * jnp.concatenate([s1[0], s1[1]], axis=1)
    p2x = di * jnp.concatenate([s2[0], s2[1]], axis=1)
    zeros = jnp.zeros((RB, 2), jnp.float32)
    rows = [
        jnp.concatenate(
            [xb[:, 2 * t:2 * t + 2], px[:, 2 * t:2 * t + 2],
             p2x[:, 2 * t:2 * t + 2], zeros], axis=1)
        for t in range(SEQ)
    ]
    out[...] = jnp.stack(rows, axis=0)


def _tc_xcat(xp, s1x, s2x, dinv):
    return pl.pallas_call(
        _xcat_body,
        grid=(NBR,),
        in_specs=[
            pl.BlockSpec((RB, 32), lambda i: (i, 0)),
            pl.BlockSpec((2, RB, 16), lambda i: (0, i, 0)),
            pl.BlockSpec((2, RB, 16), lambda i: (0, i, 0)),
            pl.BlockSpec((RB, 1), lambda i: (i, 0)),
        ],
        out_specs=pl.BlockSpec((SEQ, RB, 8), lambda i: (0, i, 0)),
        out_shape=jax.ShapeDtypeStruct((SEQ, NP, 8), jnp.float32),
    )(xp, s1x, s2x, dinv)


def _step_body(h_r, s1_r, s2_r, xc_r, dinv_r, wcat_r, b1_r, h3_r, b2_r,
               wz_r, bz_r, wr_r, br_r, wc_r, bc_r, hn_o, u_o):
    di = dinv_r[...]
    h = h_r[...]
    s1 = s1_r[...]
    s2 = s2_r[...]
    s1f = jnp.concatenate([s1[0], s1[1]], axis=1)
    s2f = jnp.concatenate([s2[0], s2[1]], axis=1)

    ic = jnp.dot(xc_r[...], wcat_r[...],
                 preferred_element_type=jnp.float32) + b1_r[...]
    hcat = jnp.concatenate([h, -di * s1f, di * s2f], axis=1)
    hc = jnp.dot(hcat, h3_r[...], preferred_element_type=jnp.float32) + b2_r[...]

    g = jnp.concatenate([ic, hc], axis=1)
    z = jax.nn.sigmoid(jnp.dot(g, wz_r[...],
                               preferred_element_type=jnp.float32) + bz_r[...])
    r = jax.nn.sigmoid(jnp.dot(g, wr_r[...],
                               preferred_element_type=jnp.float32) + br_r[...])
    cand = jnp.concatenate([ic, r * hc], axis=1)
    ht = jnp.tanh(jnp.dot(cand, wc_r[...],
                          preferred_element_type=jnp.float32) + bc_r[...])
    hn = z * h + (1.0 - z) * ht
    hn_o[...] = hn
    u_o[...] = jnp.stack([di * hn[:, :32], di * hn[:, 32:]], axis=0)


def _full(shape):
    return pl.BlockSpec(shape, lambda i: tuple(0 for _ in shape))


def _tc_step(h, s1h, s2h, xct, dinv, wcat, b1, h3, b2, wz, bz, wr, br, wc, bc):
    return pl.pallas_call(
        _step_body,
        grid=(NBR,),
        in_specs=[
            pl.BlockSpec((RB, HID), lambda i: (i, 0)),
            pl.BlockSpec((2, RB, 32), lambda i: (0, i, 0)),
            pl.BlockSpec((2, RB, 32), lambda i: (0, i, 0)),
            pl.BlockSpec((RB, 8), lambda i: (i, 0)),
            pl.BlockSpec((RB, 1), lambda i: (i, 0)),
            _full((8, HID)), _full((1, HID)),
            _full((3 * HID, HID)), _full((1, HID)),
            _full((2 * HID, HID)), _full((1, HID)),
            _full((2 * HID, HID)), _full((1, HID)),
            _full((2 * HID, HID)), _full((1, HID)),
        ],
        out_specs=[
            pl.BlockSpec((RB, HID), lambda i: (i, 0)),
            pl.BlockSpec((2, RB, 32), lambda i: (0, i, 0)),
        ],
        out_shape=[
            jax.ShapeDtypeStruct((NP, HID), jnp.float32),
            jax.ShapeDtypeStruct((2, NP, 32), jnp.float32),
        ],
    )(h, s1h, s2h, xct, dinv, wcat, b1, h3, b2, wz, bz, wr, br, wc, bc)


def _epi_body(h_r, wo_r, bo_r, out_o):
    out_o[...] = jnp.dot(h_r[...], wo_r[...],
                         preferred_element_type=jnp.float32) + bo_r[...]


def _tc_epi(h, wo, bo):
    return pl.pallas_call(
        _epi_body,
        grid=(NBR,),
        in_specs=[
            pl.BlockSpec((RB, HID), lambda i: (i, 0)),
            _full((HID, HORIZON)), _full((1, HORIZON)),
        ],
        out_specs=pl.BlockSpec((RB, HORIZON), lambda i: (i, 0)),
        out_shape=jax.ShapeDtypeStruct((NP, HORIZON), jnp.float32),
    )(h, wo, bo)


def kernel(x, edge_index, W1, b1, W2, b2, Wz, bz, Wr, br, Wc, bc, Wo, bo):
    f32 = jnp.float32
    src = edge_index[0].astype(jnp.int32)
    dst = edge_index[1].astype(jnp.int32)
    pad = EP - N_EDGES
    srcb = jnp.pad(src, (0, pad)).reshape(NBLK, BLK)
    src3 = jnp.stack([srcb, srcb + NP], axis=0)
    dstb = jnp.pad(dst, (0, pad), constant_values=GARB).reshape(NBLK, BLK)
    srcsb = jnp.pad(src, (0, pad), constant_values=GARB).reshape(NBLK, BLK)

    xp = jnp.transpose(x[0], (1, 0, 2)).reshape(N_NODES, SEQ * 2)
    xp = jnp.pad(xp, ((0, NP - N_NODES), (0, 32 - SEQ * 2)))

    wcat = jnp.concatenate(
        [W1[0] - W1[2], W1[1], 2.0 * W1[2], jnp.zeros((2, HID), f32)], axis=0)
    h3 = jnp.concatenate([W2[0] - W2[2], W2[1], 2.0 * W2[2]], axis=0)
    b1r = b1.reshape(1, HID)
    b2r = b2.reshape(1, HID)
    bzr = bz.reshape(1, HID)
    brr = br.reshape(1, HID)
    bcr = bc.reshape(1, HID)
    bor = bo.reshape(1, HORIZON)

    degs = _deg_kernel(srcsb)
    dinv, ux = _tc_prep(degs, xp)

    s1x = _spmm16(ux.reshape(2 * NP, 16), src3, dstb)
    u2x = _tc_mid(s1x, dinv, 16)
    s2x = _spmm16(u2x.reshape(2 * NP, 16), src3, dstb)

    xcat = _tc_xcat(xp, s1x, s2x, dinv)

    h = jnp.zeros((NP, HID), f32)
    zs = jnp.zeros((2, NP, 32), f32)
    u = None
    for t in range(SEQ):
        if t == 0:
            s1h, s2h = zs, zs
        else:
            s1h = _spmm32(u.reshape(2 * NP, 32), src3, dstb)
            u2h = _tc_mid(s1h, dinv, 32)
            s2h = _spmm32(u2h.reshape(2 * NP, 32), src3, dstb)
        h, u = _tc_step(h, s1h, s2h, xcat[t], dinv, wcat, b1r, h3, b2r,
                        Wz, bzr, Wr, brr, Wc, bcr)

    out = _tc_epi(h, Wo, bor)
    return out[:N_NODES].T.reshape(1, HORIZON, N_NODES)


# speedup vs baseline: 13.2013x; 1.1690x over previous
import functools

import jax
import jax.numpy as jnp
from jax import lax
from jax.experimental import pallas as pl
from jax.experimental.pallas import tpu as pltpu
from jax.experimental.pallas import tpu_sc as plsc

N_NODES = 50000
N_EDGES = 800000
HID = 64
SEQ = 12
HORIZON = 12

RB = 256
NP = 50176
NBR = NP // RB
BLK = 128
EP = 802816
NBLK = EP // BLK
N_TILES = 16
ACC_R = 51200
ROWS_PT = ACC_R // N_TILES
GARB = NP

_MESH = functools.partial(
    plsc.VectorSubcoreMesh, core_axis_name="c", subcore_axis_name="s")
_SC_PARAMS = pltpu.CompilerParams(use_tc_tiling_on_sc=False)


def _fill_rows(ref, rows, wh, value):
    vec = jnp.full((16,), value, jnp.float32)

    def body(i, _):
        for j in range(wh // 16):
            ref[i, pl.ds(16 * j, 16)] = vec
        return 0

    lax.fori_loop(0, rows, body, 0)


def _zero_acc(zrow, acc, tile):
    base = tile * ROWS_PT

    def body(i, _):
        pltpu.sync_copy(zrow, acc.at[pl.ds(base + i * BLK, BLK), :])
        return 0

    lax.fori_loop(0, ROWS_PT // BLK, body, 0)


def _copy_out(acc, out, core, tile, wh):
    base = tile * ROWS_PT
    nch = jnp.minimum(ROWS_PT // BLK, (NP - base) // BLK)

    def body(i, _):
        r0 = base + i * BLK
        pltpu.sync_copy(acc.at[pl.ds(r0, BLK), :], out.at[core, pl.ds(r0, BLK), :])
        return 0

    lax.fori_loop(0, nch, body, 0)


GRP = 14


def _make_fused(wh, grp):
    nbt = NBLK // N_TILES
    nfull, tail = divmod(nbt, grp)

    @functools.partial(
        pl.kernel,
        out_type=(
            jax.ShapeDtypeStruct((2, NP, wh), jnp.float32),
            jax.ShapeDtypeStruct((2, NP, wh), jnp.float32),
            jax.ShapeDtypeStruct((2 * NP, wh), jnp.float32),
        ),
        mesh=_MESH(),
        scratch_types=[
            pltpu.VMEM((grp, BLK), jnp.int32),
            pltpu.VMEM((grp, BLK), jnp.int32),
            pltpu.VMEM((grp * BLK, wh), jnp.float32),
            pltpu.VMEM((BLK,), jnp.float32),
            pltpu.SemaphoreType.DMA,
            pltpu.SemaphoreType.DMA,
            pltpu.VMEM_SHARED((ACC_R, wh), jnp.float32),
        ],
        compiler_params=_SC_PARAMS,
    )
    def fused(table, src3, dstb, d2, s1_o, s2_o, u2_o,
              sidx, didx, gbuf, d2v, gsem, ssem, acc):
        c = lax.axis_index("c")
        s = lax.axis_index("s")
        base = s * ROWS_PT
        nch = jnp.minimum(ROWS_PT // BLK, (NP - base) // BLK)

        def zero_and_sync():
            _fill_rows(gbuf, BLK, wh, 0.0)
            _zero_acc(gbuf.at[pl.ds(0, BLK)], acc, s)
            plsc.subcore_barrier()

        def spmm_pass(tbl):
            blk0 = s * nbt

            def run_group(row0, n):
                pltpu.sync_copy(src3.at[c, pl.ds(row0, n)],
                                sidx.at[pl.ds(0, n)])
                pltpu.sync_copy(dstb.at[pl.ds(row0, n)],
                                didx.at[pl.ds(0, n)])
                gds = [
                    pltpu.async_copy(
                        tbl.at[sidx.at[g]], gbuf.at[pl.ds(g * BLK, BLK)],
                        gsem)
                    for g in range(n)
                ]
                sds = []
                for g in range(n):
                    gds[g].wait()
                    sds.append(pltpu.async_copy(
                        gbuf.at[pl.ds(g * BLK, BLK)], acc.at[didx.at[g]],
                        ssem, add=True))
                for d in sds:
                    d.wait()

            def grp_body(gi, _):
                run_group(blk0 + gi * grp, grp)
                return 0

            lax.fori_loop(0, nfull, grp_body, 0)
            if tail:
                run_group(blk0 + nfull * grp, tail)
            plsc.subcore_barrier()

        zero_and_sync()
        spmm_pass(table)

        rbuf = gbuf.at[pl.ds(0, BLK)]

        def drain_body(i, _):
            r0 = base + i * BLK
            pltpu.sync_copy(acc.at[pl.ds(r0, BLK), :], rbuf)
            pltpu.sync_copy(rbuf, s1_o.at[c, pl.ds(r0, BLK), :])
            pltpu.sync_copy(d2.at[r0 // BLK], d2v)

            def scale_rows(k, _):
                dvec = d2v[pl.ds(16 * k, 16)]
                for r16 in range(16):
                    row = 16 * k + r16
                    dv = dvec[r16]
                    for j in range(wh // 16):
                        sl = pl.ds(16 * j, 16)
                        rbuf[row, sl] = rbuf[row, sl] * dv
                return 0

            lax.fori_loop(0, BLK // 16, scale_rows, 0)
            pltpu.sync_copy(rbuf, u2_o.at[pl.ds(c * NP + r0, BLK), :])
            return 0

        lax.fori_loop(0, nch, drain_body, 0)
        plsc.subcore_barrier()

        zero_and_sync()
        spmm_pass(u2_o)
        _copy_out(acc, s2_o, c, s, wh)

    return fused


_fused32 = _make_fused(32, 6)
_fused16 = _make_fused(16, GRP)


@functools.partial(
    pl.kernel,
    out_type=jax.ShapeDtypeStruct((2, NP, 16), jnp.float32),
    mesh=_MESH(),
    scratch_types=[
        pltpu.VMEM((GRP, BLK), jnp.int32),
        pltpu.VMEM((BLK, 16), jnp.float32),
        pltpu.VMEM((BLK, 16), jnp.float32),
        pltpu.SemaphoreType.DMA,
        pltpu.VMEM_SHARED((ACC_R, 16), jnp.float32),
    ],
    compiler_params=_SC_PARAMS,
)
def _deg_kernel(srcsb, out, didx, ones, zrow, sem, acc):
    c = lax.axis_index("c")
    s = lax.axis_index("s")
    _fill_rows(ones, BLK, 16, 1.0)
    _fill_rows(zrow, BLK, 16, 0.0)
    _zero_acc(zrow, acc, s)
    plsc.subcore_barrier()

    nbt = NBLK // (2 * N_TILES)
    blk0 = (c * N_TILES + s) * nbt

    def grp_body(gi, _):
        row0 = blk0 + gi * GRP
        pltpu.sync_copy(srcsb.at[pl.ds(row0, GRP)], didx)
        sds = [
            pltpu.async_copy(ones, acc.at[didx.at[g]], sem, add=True)
            for g in range(GRP)
        ]
        for d in sds:
            d.wait()
        return 0

    lax.fori_loop(0, nbt // GRP, grp_body, 0)
    plsc.subcore_barrier()
    _copy_out(acc, out, c, s, 16)


RB2 = 1024


def _prep_body(degs, xp, dinv_o, ux_o, d2_o):
    deg = degs[0, :, 0:1] + degs[1, :, 0:1]
    di = jnp.where(deg > 0, lax.rsqrt(deg), 0.0)
    dinv_o[...] = di
    xb = xp[...]
    ux_o[...] = jnp.stack([di * xb[:, :16], di * xb[:, 16:]], axis=0)
    d2_o[...] = (di * di).reshape(RB2 // BLK, BLK)


def _tc_prep(degs, xp):
    return pl.pallas_call(
        _prep_body,
        grid=(NP // RB2,),
        in_specs=[
            pl.BlockSpec((2, RB2, 16), lambda i: (0, i, 0)),
            pl.BlockSpec((RB2, 32), lambda i: (i, 0)),
        ],
        out_specs=[
            pl.BlockSpec((RB2, 1), lambda i: (i, 0)),
            pl.BlockSpec((2, RB2, 16), lambda i: (0, i, 0)),
            pl.BlockSpec((RB2 // BLK, BLK), lambda i: (i, 0)),
        ],
        out_shape=[
            jax.ShapeDtypeStruct((NP, 1), jnp.float32),
            jax.ShapeDtypeStruct((2, NP, 16), jnp.float32),
            jax.ShapeDtypeStruct((NP // BLK, BLK), jnp.float32),
        ],
    )(degs, xp)


def _xcat_body(xp, s1x, s2x, dinv, out):
    di = dinv[...]
    xb = xp[...]
    s1 = s1x[...]
    s2 = s2x[...]
    px = -di * jnp.concatenate([s1[0], s1[1]], axis=1)
    p2x = di * jnp.concatenate([s2[0], s2[1]], axis=1)
    zeros = jnp.zeros((RB, 2), jnp.float32)
    rows = [
        jnp.concatenate(
            [xb[:, 2 * t:2 * t + 2], px[:, 2 * t:2 * t + 2],
             p2x[:, 2 * t:2 * t + 2], zeros], axis=1)
        for t in range(SEQ)
    ]
    out[...] = jnp.stack(rows, axis=0)


def _tc_xcat(xp, s1x, s2x, dinv):
    return pl.pallas_call(
        _xcat_body,
        grid=(NBR,),
        in_specs=[
            pl.BlockSpec((RB, 32), lambda i: (i, 0)),
            pl.BlockSpec((2, RB, 16), lambda i: (0, i, 0)),
            pl.BlockSpec((2, RB, 16), lambda i: (0, i, 0)),
            pl.BlockSpec((RB, 1), lambda i: (i, 0)),
        ],
        out_specs=pl.BlockSpec((SEQ, RB, 8), lambda i: (0, i, 0)),
        out_shape=jax.ShapeDtypeStruct((SEQ, NP, 8), jnp.float32),
    )(xp, s1x, s2x, dinv)


def _step_body(h_r, s1_r, s2_r, xc_r, dinv_r, wcat_r, b1_r, h3_r, b2_r,
               wz_r, bz_r, wr_r, br_r, wc_r, bc_r, hn_o, u_o):
    di = dinv_r[...]
    h = h_r[...]
    s1 = s1_r[...]
    s2 = s2_r[...]
    s1f = jnp.concatenate([s1[0], s1[1]], axis=1)
    s2f = jnp.concatenate([s2[0], s2[1]], axis=1)

    ic = jnp.dot(xc_r[...], wcat_r[...],
                 preferred_element_type=jnp.float32) + b1_r[...]
    hcat = jnp.concatenate([h, -di * s1f, di * s2f], axis=1)
    hc = jnp.dot(hcat, h3_r[...], preferred_element_type=jnp.float32) + b2_r[...]

    g = jnp.concatenate([ic, hc], axis=1)
    z = jax.nn.sigmoid(jnp.dot(g, wz_r[...],
                               preferred_element_type=jnp.float32) + bz_r[...])
    r = jax.nn.sigmoid(jnp.dot(g, wr_r[...],
                               preferred_element_type=jnp.float32) + br_r[...])
    cand = jnp.concatenate([ic, r * hc], axis=1)
    ht = jnp.tanh(jnp.dot(cand, wc_r[...],
                          preferred_element_type=jnp.float32) + bc_r[...])
    hn = z * h + (1.0 - z) * ht
    hn_o[...] = hn
    u_o[...] = jnp.stack([di * hn[:, :32], di * hn[:, 32:]], axis=0)


def _full(shape):
    return pl.BlockSpec(shape, lambda i: tuple(0 for _ in shape))


def _tc_step(h, s1h, s2h, xct, dinv, wcat, b1, h3, b2, wz, bz, wr, br, wc, bc):
    return pl.pallas_call(
        _step_body,
        grid=(NBR,),
        in_specs=[
            pl.BlockSpec((RB, HID), lambda i: (i, 0)),
            pl.BlockSpec((2, RB, 32), lambda i: (0, i, 0)),
            pl.BlockSpec((2, RB, 32), lambda i: (0, i, 0)),
            pl.BlockSpec((RB, 8), lambda i: (i, 0)),
            pl.BlockSpec((RB, 1), lambda i: (i, 0)),
            _full((8, HID)), _full((1, HID)),
            _full((3 * HID, HID)), _full((1, HID)),
            _full((2 * HID, HID)), _full((1, HID)),
            _full((2 * HID, HID)), _full((1, HID)),
            _full((2 * HID, HID)), _full((1, HID)),
        ],
        out_specs=[
            pl.BlockSpec((RB, HID), lambda i: (i, 0)),
            pl.BlockSpec((2, RB, 32), lambda i: (0, i, 0)),
        ],
        out_shape=[
            jax.ShapeDtypeStruct((NP, HID), jnp.float32),
            jax.ShapeDtypeStruct((2, NP, 32), jnp.float32),
        ],
    )(h, s1h, s2h, xct, dinv, wcat, b1, h3, b2, wz, bz, wr, br, wc, bc)


def _epi_body(h_r, wo_r, bo_r, out_o):
    out_o[...] = jnp.dot(h_r[...], wo_r[...],
                         preferred_element_type=jnp.float32) + bo_r[...]


def _tc_epi(h, wo, bo):
    return pl.pallas_call(
        _epi_body,
        grid=(NBR,),
        in_specs=[
            pl.BlockSpec((RB, HID), lambda i: (i, 0)),
            _full((HID, HORIZON)), _full((1, HORIZON)),
        ],
        out_specs=pl.BlockSpec((RB, HORIZON), lambda i: (i, 0)),
        out_shape=jax.ShapeDtypeStruct((NP, HORIZON), jnp.float32),
    )(h, wo, bo)


def kernel(x, edge_index, W1, b1, W2, b2, Wz, bz, Wr, br, Wc, bc, Wo, bo):
    f32 = jnp.float32
    src = edge_index[0].astype(jnp.int32)
    dst = edge_index[1].astype(jnp.int32)
    pad = EP - N_EDGES
    srcb = jnp.pad(src, (0, pad)).reshape(NBLK, BLK)
    src3 = jnp.stack([srcb, srcb + NP], axis=0)
    dstb = jnp.pad(dst, (0, pad), constant_values=GARB).reshape(NBLK, BLK)
    srcsb = jnp.pad(src, (0, pad), constant_values=GARB).reshape(NBLK, BLK)

    xp = jnp.transpose(x[0], (1, 0, 2)).reshape(N_NODES, SEQ * 2)
    xp = jnp.pad(xp, ((0, NP - N_NODES), (0, 32 - SEQ * 2)))

    wcat = jnp.concatenate(
        [W1[0] - W1[2], W1[1], 2.0 * W1[2], jnp.zeros((2, HID), f32)], axis=0)
    h3 = jnp.concatenate([W2[0] - W2[2], W2[1], 2.0 * W2[2]], axis=0)
    b1r = b1.reshape(1, HID)
    b2r = b2.reshape(1, HID)
    bzr = bz.reshape(1, HID)
    brr = br.reshape(1, HID)
    bcr = bc.reshape(1, HID)
    bor = bo.reshape(1, HORIZON)

    degs = _deg_kernel(srcsb)
    dinv, ux, d2 = _tc_prep(degs, xp)

    s1x, s2x, _ = _fused16(ux.reshape(2 * NP, 16), src3, dstb, d2)

    xcat = _tc_xcat(xp, s1x, s2x, dinv)

    h = jnp.zeros((NP, HID), f32)
    zs = jnp.zeros((2, NP, 32), f32)
    u = None
    for t in range(SEQ):
        if t == 0:
            s1h, s2h = zs, zs
        else:
            s1h, s2h, _ = _fused32(u.reshape(2 * NP, 32), src3, dstb, d2)
        h, u = _tc_step(h, s1h, s2h, xcat[t], dinv, wcat, b1r, h3, b2r,
                        Wz, bzr, Wr, brr, Wc, bcr)

    out = _tc_epi(h, Wo, bor)
    return out[:N_NODES].T.reshape(1, HORIZON, N_NODES)


# speedup vs baseline: 14.5150x; 1.0995x over previous
import functools

import jax
import jax.numpy as jnp
from jax import lax
from jax.experimental import pallas as pl
from jax.experimental.pallas import tpu as pltpu
from jax.experimental.pallas import tpu_sc as plsc

N_NODES = 50000
N_EDGES = 800000
HID = 64
SEQ = 12
HORIZON = 12

RB = 256
NP = 50176
NBR = NP // RB
BLK = 128
EP = 802816
NBLK = EP // BLK
N_TILES = 16
ACC_R = 51200
ROWS_PT = ACC_R // N_TILES
GARB = NP

_MESH = functools.partial(
    plsc.VectorSubcoreMesh, core_axis_name="c", subcore_axis_name="s")
_SC_PARAMS = pltpu.CompilerParams(use_tc_tiling_on_sc=False)


def _fill_rows(ref, rows, wh, value):
    vec = jnp.full((16,), value, jnp.float32)

    def body(i, _):
        for j in range(wh // 16):
            ref[i, pl.ds(16 * j, 16)] = vec
        return 0

    lax.fori_loop(0, rows, body, 0)


def _zero_acc(zrow, acc, tile):
    base = tile * ROWS_PT

    def body(i, _):
        pltpu.sync_copy(zrow, acc.at[pl.ds(base + i * BLK, BLK), :])
        return 0

    lax.fori_loop(0, ROWS_PT // BLK, body, 0)


def _copy_out(acc, out, core, tile, wh):
    base = tile * ROWS_PT
    nch = jnp.minimum(ROWS_PT // BLK, (NP - base) // BLK)

    def body(i, _):
        r0 = base + i * BLK
        pltpu.sync_copy(acc.at[pl.ds(r0, BLK), :], out.at[core, pl.ds(r0, BLK), :])
        return 0

    lax.fori_loop(0, nch, body, 0)


GRP = 14


def _make_fused(wh, grp):
    nbt = NBLK // N_TILES
    slab = 2 * grp
    nslab, stail = divmod(nbt, slab)
    WAVE = 6
    NCHT = ROWS_PT // BLK

    @functools.partial(
        pl.kernel,
        out_type=(
            jax.ShapeDtypeStruct((2, NP, wh), jnp.float32),
            jax.ShapeDtypeStruct((2, NP, wh), jnp.float32),
            jax.ShapeDtypeStruct((2 * NP, wh), jnp.float32),
        ),
        mesh=_MESH(),
        scratch_types=[
            pltpu.VMEM((slab, BLK), jnp.int32),
            pltpu.VMEM((slab, BLK), jnp.int32),
            pltpu.VMEM((grp * BLK, wh), jnp.float32),
            pltpu.VMEM((WAVE, BLK), jnp.float32),
            pltpu.SemaphoreType.DMA,
            pltpu.SemaphoreType.DMA,
            pltpu.SemaphoreType.DMA,
            pltpu.VMEM_SHARED((ACC_R, wh), jnp.float32),
        ],
        compiler_params=_SC_PARAMS,
    )
    def fused(table, src3, dstb, d2, s1_o, s2_o, u2_o,
              sidx, didx, gbuf, d2buf, gsem, ssem, osem, acc):
        c = lax.axis_index("c")
        s = lax.axis_index("s")
        base = s * ROWS_PT
        nch = jnp.minimum(NCHT, (NP - base) // BLK)
        zsrc = gbuf.at[pl.ds(0, BLK)]

        def chunk(i):
            return pl.ds(base + i * BLK, BLK)

        def zero_acc_async():
            def fire(i, _):
                pltpu.async_copy(zsrc, acc.at[chunk(i), :], osem)
                return 0

            def drain(i, _):
                pltpu.make_async_copy(zsrc, acc.at[chunk(i), :], osem).wait()
                return 0

            lax.fori_loop(0, NCHT, fire, 0)
            lax.fori_loop(0, NCHT, drain, 0)

        def spmm_pass(tbl):
            blk0 = s * nbt

            def run_groups(row0, nblks):
                pltpu.sync_copy(src3.at[c, pl.ds(row0, nblks)],
                                sidx.at[pl.ds(0, nblks)])
                pltpu.sync_copy(dstb.at[pl.ds(row0, nblks)],
                                didx.at[pl.ds(0, nblks)])
                for g0 in range(0, nblks, grp):
                    n = min(grp, nblks - g0)
                    gds = [
                        pltpu.async_copy(
                            tbl.at[sidx.at[g0 + g]],
                            gbuf.at[pl.ds(g * BLK, BLK)], gsem)
                        for g in range(n)
                    ]
                    sds = []
                    for g in range(n):
                        gds[g].wait()
                        sds.append(pltpu.async_copy(
                            gbuf.at[pl.ds(g * BLK, BLK)],
                            acc.at[didx.at[g0 + g]], ssem, add=True))
                    for d in sds:
                        d.wait()

            def slab_body(gi, _):
                run_groups(blk0 + gi * slab, slab)
                return 0

            lax.fori_loop(0, nslab, slab_body, 0)
            if stail:
                run_groups(blk0 + nslab * slab, stail)
            plsc.subcore_barrier()

        def scale_chunk(slot, d2row, kbase):
            def sk(k, _):
                dvec = d2buf[d2row, pl.ds(16 * k, 16)]
                for r16 in range(16):
                    row = slot * BLK + 16 * k + r16
                    dv = dvec[r16]
                    for j in range(wh // 16):
                        sl = pl.ds(16 * j, 16)
                        gbuf[row, sl] = gbuf[row, sl] * dv
                return 0

            lax.fori_loop(0, BLK // 16, sk, 0)
            del kbase

        _fill_rows(zsrc, BLK, wh, 0.0)
        zero_acc_async()
        plsc.subcore_barrier()
        spmm_pass(table)

        def s1_fire(i, _):
            pltpu.async_copy(acc.at[chunk(i), :],
                             s1_o.at[c, chunk(i), :], osem)
            return 0

        lax.fori_loop(0, nch, s1_fire, 0)

        nw = nch // WAVE

        def wave_body(w, _):
            k0 = w * WAVE
            gds = [
                pltpu.async_copy(acc.at[chunk(k0 + j), :],
                                 gbuf.at[pl.ds(j * BLK, BLK)], gsem)
                for j in range(WAVE)
            ]
            pltpu.sync_copy(d2.at[pl.ds(base // BLK + k0, WAVE)], d2buf)
            sds = []
            for j in range(WAVE):
                gds[j].wait()
                scale_chunk(j, j, k0)
                sds.append(pltpu.async_copy(
                    gbuf.at[pl.ds(j * BLK, BLK)],
                    u2_o.at[pl.ds(c * NP + base + (k0 + j) * BLK, BLK), :],
                    ssem))
            for d in sds:
                d.wait()
            return 0

        lax.fori_loop(0, nw, wave_body, 0)

        def tail_body(i, _):
            k = nw * WAVE + i
            pltpu.sync_copy(acc.at[chunk(k), :], zsrc)
            pltpu.sync_copy(d2.at[pl.ds(base // BLK + k, 1)],
                            d2buf.at[pl.ds(0, 1)])
            scale_chunk(0, 0, k)
            pltpu.sync_copy(zsrc,
                            u2_o.at[pl.ds(c * NP + base + k * BLK, BLK), :])
            return 0

        lax.fori_loop(0, nch - nw * WAVE, tail_body, 0)

        def s1_drain(i, _):
            pltpu.make_async_copy(acc.at[chunk(i), :],
                                  s1_o.at[c, chunk(i), :], osem).wait()
            return 0

        lax.fori_loop(0, nch, s1_drain, 0)
        plsc.subcore_barrier()

        _fill_rows(zsrc, BLK, wh, 0.0)
        zero_acc_async()
        plsc.subcore_barrier()
        spmm_pass(u2_o)

        def s2_fire(i, _):
            pltpu.async_copy(acc.at[chunk(i), :],
                             s2_o.at[c, chunk(i), :], osem)
            return 0

        def s2_drain(i, _):
            pltpu.make_async_copy(acc.at[chunk(i), :],
                                  s2_o.at[c, chunk(i), :], osem).wait()
            return 0

        lax.fori_loop(0, nch, s2_fire, 0)
        lax.fori_loop(0, nch, s2_drain, 0)

    return fused


_fused32 = _make_fused(32, 6)
_fused16 = _make_fused(16, GRP)


@functools.partial(
    pl.kernel,
    out_type=jax.ShapeDtypeStruct((2, NP, 16), jnp.float32),
    mesh=_MESH(),
    scratch_types=[
        pltpu.VMEM((GRP, BLK), jnp.int32),
        pltpu.VMEM((BLK, 16), jnp.float32),
        pltpu.VMEM((BLK, 16), jnp.float32),
        pltpu.SemaphoreType.DMA,
        pltpu.VMEM_SHARED((ACC_R, 16), jnp.float32),
    ],
    compiler_params=_SC_PARAMS,
)
def _deg_kernel(srcsb, out, didx, ones, zrow, sem, acc):
    c = lax.axis_index("c")
    s = lax.axis_index("s")
    _fill_rows(ones, BLK, 16, 1.0)
    _fill_rows(zrow, BLK, 16, 0.0)
    _zero_acc(zrow, acc, s)
    plsc.subcore_barrier()

    nbt = NBLK // (2 * N_TILES)
    blk0 = (c * N_TILES + s) * nbt

    def grp_body(gi, _):
        row0 = blk0 + gi * GRP
        pltpu.sync_copy(srcsb.at[pl.ds(row0, GRP)], didx)
        sds = [
            pltpu.async_copy(ones, acc.at[didx.at[g]], sem, add=True)
            for g in range(GRP)
        ]
        for d in sds:
            d.wait()
        return 0

    lax.fori_loop(0, nbt // GRP, grp_body, 0)
    plsc.subcore_barrier()
    _copy_out(acc, out, c, s, 16)


RB2 = 1024


def _prep_body(degs, xp, dinv_o, ux_o, d2_o):
    deg = degs[0, :, 0:1] + degs[1, :, 0:1]
    di = jnp.where(deg > 0, lax.rsqrt(deg), 0.0)
    dinv_o[...] = di
    xb = xp[...]
    ux_o[...] = jnp.stack([di * xb[:, :16], di * xb[:, 16:]], axis=0)
    d2_o[...] = (di * di).reshape(RB2 // BLK, BLK)


def _tc_prep(degs, xp):
    return pl.pallas_call(
        _prep_body,
        grid=(NP // RB2,),
        in_specs=[
            pl.BlockSpec((2, RB2, 16), lambda i: (0, i, 0)),
            pl.BlockSpec((RB2, 32), lambda i: (i, 0)),
        ],
        out_specs=[
            pl.BlockSpec((RB2, 1), lambda i: (i, 0)),
            pl.BlockSpec((2, RB2, 16), lambda i: (0, i, 0)),
            pl.BlockSpec((RB2 // BLK, BLK), lambda i: (i, 0)),
        ],
        out_shape=[
            jax.ShapeDtypeStruct((NP, 1), jnp.float32),
            jax.ShapeDtypeStruct((2, NP, 16), jnp.float32),
            jax.ShapeDtypeStruct((NP // BLK, BLK), jnp.float32),
        ],
    )(degs, xp)


def _xcat_body(xp, s1x, s2x, dinv, out):
    di = dinv[...]
    xb = xp[...]
    s1 = s1x[...]
    s2 = s2x[...]
    px = -di * jnp.concatenate([s1[0], s1[1]], axis=1)
    p2x = di * jnp.concatenate([s2[0], s2[1]], axis=1)
    zeros = jnp.zeros((RB, 2), jnp.float32)
    rows = [
        jnp.concatenate(
            [xb[:, 2 * t:2 * t + 2], px[:, 2 * t:2 * t + 2],
             p2x[:, 2 * t:2 * t + 2], zeros], axis=1)
        for t in range(SEQ)
    ]
    out[...] = jnp.stack(rows, axis=0)


def _tc_xcat(xp, s1x, s2x, dinv):
    return pl.pallas_call(
        _xcat_body,
        grid=(NBR,),
        in_specs=[
            pl.BlockSpec((RB, 32), lambda i: (i, 0)),
            pl.BlockSpec((2, RB, 16), lambda i: (0, i, 0)),
            pl.BlockSpec((2, RB, 16), lambda i: (0, i, 0)),
            pl.BlockSpec((RB, 1), lambda i: (i, 0)),
        ],
        out_specs=pl.BlockSpec((SEQ, RB, 8), lambda i: (0, i, 0)),
        out_shape=jax.ShapeDtypeStruct((SEQ, NP, 8), jnp.float32),
    )(xp, s1x, s2x, dinv)


def _step_body(h_r, s1_r, s2_r, xc_r, dinv_r, wcat_r, b1_r, h3_r, b2_r,
               wz_r, bz_r, wr_r, br_r, wc_r, bc_r, hn_o, u_o):
    di = dinv_r[...]
    h = h_r[...]
    s1 = s1_r[...]
    s2 = s2_r[...]
    s1f = jnp.concatenate([s1[0], s1[1]], axis=1)
    s2f = jnp.concatenate([s2[0], s2[1]], axis=1)

    ic = jnp.dot(xc_r[...], wcat_r[...],
                 preferred_element_type=jnp.float32) + b1_r[...]
    hcat = jnp.concatenate([h, -di * s1f, di * s2f], axis=1)
    hc = jnp.dot(hcat, h3_r[...], preferred_element_type=jnp.float32) + b2_r[...]

    g = jnp.concatenate([ic, hc], axis=1)
    z = jax.nn.sigmoid(jnp.dot(g, wz_r[...],
                               preferred_element_type=jnp.float32) + bz_r[...])
    r = jax.nn.sigmoid(jnp.dot(g, wr_r[...],
                               preferred_element_type=jnp.float32) + br_r[...])
    cand = jnp.concatenate([ic, r * hc], axis=1)
    ht = jnp.tanh(jnp.dot(cand, wc_r[...],
                          preferred_element_type=jnp.float32) + bc_r[...])
    hn = z * h + (1.0 - z) * ht
    hn_o[...] = hn
    u_o[...] = jnp.stack([di * hn[:, :32], di * hn[:, 32:]], axis=0)


def _full(shape):
    return pl.BlockSpec(shape, lambda i: tuple(0 for _ in shape))


def _tc_step(h, s1h, s2h, xct, dinv, wcat, b1, h3, b2, wz, bz, wr, br, wc, bc):
    return pl.pallas_call(
        _step_body,
        grid=(NBR,),
        in_specs=[
            pl.BlockSpec((RB, HID), lambda i: (i, 0)),
            pl.BlockSpec((2, RB, 32), lambda i: (0, i, 0)),
            pl.BlockSpec((2, RB, 32), lambda i: (0, i, 0)),
            pl.BlockSpec((RB, 8), lambda i: (i, 0)),
            pl.BlockSpec((RB, 1), lambda i: (i, 0)),
            _full((8, HID)), _full((1, HID)),
            _full((3 * HID, HID)), _full((1, HID)),
            _full((2 * HID, HID)), _full((1, HID)),
            _full((2 * HID, HID)), _full((1, HID)),
            _full((2 * HID, HID)), _full((1, HID)),
        ],
        out_specs=[
            pl.BlockSpec((RB, HID), lambda i: (i, 0)),
            pl.BlockSpec((2, RB, 32), lambda i: (0, i, 0)),
        ],
        out_shape=[
            jax.ShapeDtypeStruct((NP, HID), jnp.float32),
            jax.ShapeDtypeStruct((2, NP, 32), jnp.float32),
        ],
    )(h, s1h, s2h, xct, dinv, wcat, b1, h3, b2, wz, bz, wr, br, wc, bc)


def _epi_body(h_r, wo_r, bo_r, out_o):
    out_o[...] = jnp.dot(h_r[...], wo_r[...],
                         preferred_element_type=jnp.float32) + bo_r[...]


def _tc_epi(h, wo, bo):
    return pl.pallas_call(
        _epi_body,
        grid=(NBR,),
        in_specs=[
            pl.BlockSpec((RB, HID), lambda i: (i, 0)),
            _full((HID, HORIZON)), _full((1, HORIZON)),
        ],
        out_specs=pl.BlockSpec((RB, HORIZON), lambda i: (i, 0)),
        out_shape=jax.ShapeDtypeStruct((NP, HORIZON), jnp.float32),
    )(h, wo, bo)


def kernel(x, edge_index, W1, b1, W2, b2, Wz, bz, Wr, br, Wc, bc, Wo, bo):
    f32 = jnp.float32
    src = edge_index[0].astype(jnp.int32)
    dst = edge_index[1].astype(jnp.int32)
    pad = EP - N_EDGES
    srcb = jnp.pad(src, (0, pad)).reshape(NBLK, BLK)
    src3 = jnp.stack([srcb, srcb + NP], axis=0)
    dstb = jnp.pad(dst, (0, pad), constant_values=GARB).reshape(NBLK, BLK)
    srcsb = jnp.pad(src, (0, pad), constant_values=GARB).reshape(NBLK, BLK)

    xp = jnp.transpose(x[0], (1, 0, 2)).reshape(N_NODES, SEQ * 2)
    xp = jnp.pad(xp, ((0, NP - N_NODES), (0, 32 - SEQ * 2)))

    wcat = jnp.concatenate(
        [W1[0] - W1[2], W1[1], 2.0 * W1[2], jnp.zeros((2, HID), f32)], axis=0)
    h3 = jnp.concatenate([W2[0] - W2[2], W2[1], 2.0 * W2[2]], axis=0)
    b1r = b1.reshape(1, HID)
    b2r = b2.reshape(1, HID)
    bzr = bz.reshape(1, HID)
    brr = br.reshape(1, HID)
    bcr = bc.reshape(1, HID)
    bor = bo.reshape(1, HORIZON)

    degs = _deg_kernel(srcsb)
    dinv, ux, d2 = _tc_prep(degs, xp)

    s1x, s2x, _ = _fused16(ux.reshape(2 * NP, 16), src3, dstb, d2)

    xcat = _tc_xcat(xp, s1x, s2x, dinv)

    h = jnp.zeros((NP, HID), f32)
    zs = jnp.zeros((2, NP, 32), f32)
    u = None
    for t in range(SEQ):
        if t == 0:
            s1h, s2h = zs, zs
        else:
            s1h, s2h, _ = _fused32(u.reshape(2 * NP, 32), src3, dstb, d2)
        h, u = _tc_step(h, s1h, s2h, xcat[t], dinv, wcat, b1r, h3, b2r,
                        Wz, bzr, Wr, brr, Wc, bcr)

    out = _tc_epi(h, Wo, bor)
    return out[:N_NODES].T.reshape(1, HORIZON, N_NODES)
